# Initial kernel scaffold; baseline (speedup 1.0000x reference)
import jax
import jax.numpy as jnp
from jax.experimental import pallas as pl


def kernel(time_interval, pe):
    raise NotImplementedError("write your pallas kernel here")


# speedup vs baseline: 7.3944x; 7.3944x over previous
import functools

import jax
import jax.numpy as jnp
from jax import lax
from jax.experimental import pallas as pl
from jax.experimental.pallas import tpu as pltpu
from jax.experimental.pallas import tpu_sc as plsc

D_MODEL = 128
NUM_WORKERS = 32
CHUNK = 128


def _gather_body(n_per_w, n_chunks, tab_hbm, idx_hbm, out_hbm,
                 idx_v, buf0, buf1, gsem0, gsem1, ssem0, ssem1):
    wid = lax.axis_index("s") * 2 + lax.axis_index("c")
    base = wid * n_per_w

    pltpu.sync_copy(idx_hbm.at[pl.ds(base, n_per_w)], idx_v)

    bufs = (buf0, buf1)
    gsems = (gsem0, gsem1)
    ssems = (ssem0, ssem1)

    def issue_gather(c, slot):
        pltpu.async_copy(
            tab_hbm.at[idx_v.at[pl.ds(c * CHUNK, CHUNK)]], bufs[slot],
            gsems[slot])

    def issue_scatter(c, slot):
        pltpu.async_copy(
            bufs[slot], out_hbm.at[pl.ds(base + c * CHUNK, CHUNK)],
            ssems[slot])

    issue_gather(0, 0)
    issue_gather(1, 1)

    def step(i, _):
        c = i * 2
        for slot in range(2):
            pltpu.make_async_copy(
                tab_hbm.at[idx_v.at[pl.ds(0, CHUNK)]], bufs[slot],
                gsems[slot]).wait()
            issue_scatter(c + slot, slot)
            nxt = c + slot + 2

            @pl.when(nxt < n_chunks)
            def _():
                pltpu.make_async_copy(
                    bufs[slot], out_hbm.at[pl.ds(base, CHUNK)],
                    ssems[slot]).wait()
                issue_gather(nxt, slot)

        return ()

    lax.fori_loop(0, n_chunks // 2, step, (), unroll=False)

    for slot in range(2):
        pltpu.make_async_copy(
            bufs[slot], out_hbm.at[pl.ds(base, CHUNK)], ssems[slot]).wait()


def _make_sc_gather(n_rows):
    n_per_w = n_rows // NUM_WORKERS
    n_chunks = n_per_w // CHUNK
    mesh = plsc.VectorSubcoreMesh(core_axis_name="c", subcore_axis_name="s")
    return pl.kernel(
        functools.partial(_gather_body, n_per_w, n_chunks),
        out_type=jax.ShapeDtypeStruct((n_rows, D_MODEL), jnp.float32),
        mesh=mesh,
        scratch_types=[
            pltpu.VMEM((n_per_w,), jnp.int32),
            pltpu.VMEM((CHUNK, D_MODEL), jnp.float32),
            pltpu.VMEM((CHUNK, D_MODEL), jnp.float32),
            pltpu.SemaphoreType.DMA,
            pltpu.SemaphoreType.DMA,
            pltpu.SemaphoreType.DMA,
            pltpu.SemaphoreType.DMA,
        ],
        name="sc_embedding_gather",
    )


@jax.jit
def kernel(time_interval, pe):
    b, t = time_interval.shape
    idx_flat = time_interval.reshape(-1).astype(jnp.int32)
    out = _make_sc_gather(b * t)(pe, idx_flat)
    return out.reshape(b, t, D_MODEL)


# speedup vs baseline: 7.5968x; 1.0274x over previous
import functools

import jax
import jax.numpy as jnp
from jax import lax
from jax.experimental import pallas as pl
from jax.experimental.pallas import tpu as pltpu
from jax.experimental.pallas import tpu_sc as plsc

D_MODEL = 128
NUM_WORKERS = 32
CHUNK = 128


NBUF = 5
LEAD = 3


def _gather_body(n_per_w, n_chunks, tab_hbm, idx_hbm, out_hbm,
                 idx_v, bufs, gsems, ssems):
    wid = lax.axis_index("s") * 2 + lax.axis_index("c")
    base = wid * n_per_w

    pltpu.sync_copy(idx_hbm.at[pl.ds(base, n_per_w)], idx_v)

    def issue_gather(c, slot):
        pltpu.async_copy(
            tab_hbm.at[idx_v.at[pl.ds(c * CHUNK, CHUNK)]], bufs[slot],
            gsems[slot])

    def issue_scatter(c, slot):
        pltpu.async_copy(
            bufs[slot], out_hbm.at[pl.ds(base + c * CHUNK, CHUNK)],
            ssems[slot])

    def wait_gather(slot):
        pltpu.make_async_copy(
            tab_hbm.at[idx_v.at[pl.ds(0, CHUNK)]], bufs[slot],
            gsems[slot]).wait()

    def wait_scatter(slot):
        pltpu.make_async_copy(
            bufs[slot], out_hbm.at[pl.ds(base, CHUNK)],
            ssems[slot]).wait()

    for c in range(LEAD):
        issue_gather(c, c)

    def step(i, _):
        for s in range(NBUF):
            c = i * NBUF + s
            pre = c + LEAD
            pre_slot = (s + LEAD) % NBUF

            @pl.when(pre < n_chunks)
            def _():
                @pl.when(pre >= NBUF)
                def _():
                    wait_scatter(pre_slot)
                issue_gather(pre, pre_slot)

            wait_gather(s)
            issue_scatter(c, s)
        return ()

    lax.fori_loop(0, n_chunks // NBUF, step, (), unroll=False)

    for slot in range(NBUF):
        wait_scatter(slot)


def _make_sc_gather(n_rows):
    n_per_w = n_rows // NUM_WORKERS
    n_chunks = n_per_w // CHUNK
    mesh = plsc.VectorSubcoreMesh(core_axis_name="c", subcore_axis_name="s")
    return pl.kernel(
        functools.partial(_gather_body, n_per_w, n_chunks),
        out_type=jax.ShapeDtypeStruct((n_rows, D_MODEL), jnp.float32),
        mesh=mesh,
        scratch_types=[
            pltpu.VMEM((n_per_w,), jnp.int32),
            [pltpu.VMEM((CHUNK, D_MODEL), jnp.float32)
             for _ in range(NBUF)],
            [pltpu.SemaphoreType.DMA for _ in range(NBUF)],
            [pltpu.SemaphoreType.DMA for _ in range(NBUF)],
        ],
        name="sc_embedding_gather",
    )


@jax.jit
def kernel(time_interval, pe):
    b, t = time_interval.shape
    idx_flat = time_interval.reshape(-1).astype(jnp.int32)
    out = _make_sc_gather(b * t)(pe, idx_flat)
    return out.reshape(b, t, D_MODEL)


# speedup vs baseline: 11.3224x; 1.4904x over previous
import functools

import jax
import jax.numpy as jnp
from jax import lax
from jax.experimental import pallas as pl
from jax.experimental.pallas import tpu as pltpu
from jax.experimental.pallas import tpu_sc as plsc

D_MODEL = 128
NUM_WORKERS = 32
CHUNK = 64


NBUF = 5
LEAD = 3


def _gather_body(n_per_w, n_chunks, tab_rows, tab_hbm, idx_hbm, out_hbm,
                 idx_v, tab_sh, bufs, gsems, ssems):
    sub = lax.axis_index("s")
    wid = sub * 2 + lax.axis_index("c")
    base = wid * n_per_w

    rows_per_sub = tab_rows // 16
    pltpu.sync_copy(tab_hbm.at[pl.ds(sub * rows_per_sub, rows_per_sub)],
                    tab_sh.at[pl.ds(sub * rows_per_sub, rows_per_sub)])
    pltpu.sync_copy(idx_hbm.at[pl.ds(base, n_per_w)], idx_v)
    plsc.subcore_barrier()

    def issue_gather(c, slot):
        pltpu.async_copy(
            tab_sh.at[idx_v.at[pl.ds(c * CHUNK, CHUNK)]], bufs[slot],
            gsems[slot])

    def issue_scatter(c, slot):
        pltpu.async_copy(
            bufs[slot], out_hbm.at[pl.ds(base + c * CHUNK, CHUNK)],
            ssems[slot])

    def wait_gather(slot):
        pltpu.make_async_copy(
            tab_hbm.at[idx_v.at[pl.ds(0, CHUNK)]], bufs[slot],
            gsems[slot]).wait()

    def wait_scatter(slot):
        pltpu.make_async_copy(
            bufs[slot], out_hbm.at[pl.ds(base, CHUNK)],
            ssems[slot]).wait()

    for c in range(LEAD):
        issue_gather(c, c)

    def step(i, _):
        for s in range(NBUF):
            c = i * NBUF + s
            pre = c + LEAD
            pre_slot = (s + LEAD) % NBUF

            @pl.when(pre < n_chunks)
            def _():
                @pl.when(pre >= NBUF)
                def _():
                    wait_scatter(pre_slot)
                issue_gather(pre, pre_slot)

            wait_gather(s)
            issue_scatter(c, s)
        return ()

    lax.fori_loop(0, n_chunks // NBUF, step, (), unroll=False)

    for slot in range(NBUF):
        wait_scatter(slot)


def _make_sc_gather(n_rows, tab_rows):
    n_per_w = n_rows // NUM_WORKERS
    n_chunks = n_per_w // CHUNK
    mesh = plsc.VectorSubcoreMesh(core_axis_name="c", subcore_axis_name="s")
    return pl.kernel(
        functools.partial(_gather_body, n_per_w, n_chunks, tab_rows),
        out_type=jax.ShapeDtypeStruct((n_rows, D_MODEL), jnp.float32),
        mesh=mesh,
        scratch_types=[
            pltpu.VMEM((n_per_w,), jnp.int32),
            pltpu.VMEM_SHARED((tab_rows, D_MODEL), jnp.float32),
            [pltpu.VMEM((CHUNK, D_MODEL), jnp.float32)
             for _ in range(NBUF)],
            [pltpu.SemaphoreType.DMA for _ in range(NBUF)],
            [pltpu.SemaphoreType.DMA for _ in range(NBUF)],
        ],
        name="sc_embedding_gather",
    )


@jax.jit
def kernel(time_interval, pe):
    b, t = time_interval.shape
    idx_flat = time_interval.reshape(-1).astype(jnp.int32)
    out = _make_sc_gather(b * t, pe.shape[0])(pe, idx_flat)
    return out.reshape(b, t, D_MODEL)
